# baseline (speedup 1.0000x reference)
import functools

import jax
import jax.numpy as jnp
from jax import lax
from jax.experimental import pallas as pl
from jax.experimental.pallas import tpu as pltpu
from jax.experimental.pallas import tpu_sc as plsc

N = 10000
E = 320000
D = 128
NP = 10240
NC = 2
NS = 16
NW = NC * NS
K = 125
RPW = 80
EPW = E // NW
BLK = 1024
GRID = NP // BLK
ZROWS = 16
GROUPS = 2
GROW = RPW // GROUPS
NEG_SLOPE = 0.2


def _sc_mesh():
    return plsc.VectorSubcoreMesh(core_axis_name="c", subcore_axis_name="s")


@functools.partial(
    pl.kernel,
    out_type=jax.ShapeDtypeStruct((NW, NP), jnp.float32),
    mesh=_sc_mesh(),
    scratch_types=[
        pltpu.VMEM((NP,), jnp.float32),
        pltpu.VMEM((EPW // 16, 16), jnp.int32),
    ],
    compiler_params=pltpu.CompilerParams(needs_layout_passes=False),
)
def _sc_degree(e3_hbm, out_hbm, deg_v, idx_v):
    c = lax.axis_index("c")
    s = lax.axis_index("s")
    w = c * NS + s
    zeros16 = jnp.zeros((16,), jnp.float32)
    ones16 = jnp.ones((16,), jnp.float32)

    def zero_body(i, _):
        deg_v[pl.ds(i * 16, 16)] = zeros16
        return ()

    lax.fori_loop(0, NP // 16, zero_body, (), unroll=4)

    pltpu.sync_copy(e3_hbm.at[1, w], idx_v)

    def hist_body(i, _):
        idx = idx_v[i]
        plsc.addupdate_scatter(deg_v, [idx], ones16)
        return ()

    lax.fori_loop(0, EPW // 16, hist_body, (), unroll=4)

    pltpu.sync_copy(deg_v, out_hbm.at[w])


@functools.partial(
    pl.kernel,
    out_type=jax.ShapeDtypeStruct((NC, NP, D), jnp.float32),
    mesh=_sc_mesh(),
    scratch_types=[
        pltpu.VMEM((GROW, K), jnp.int32),
        pltpu.VMEM((GROW, K), jnp.int32),
        pltpu.VMEM((K, D), jnp.float32),
        pltpu.VMEM((K, D), jnp.float32),
        pltpu.VMEM((ZROWS, D), jnp.float32),
        pltpu.VMEM_SHARED((NP, D), jnp.float32),
        pltpu.SemaphoreType.DMA,
        pltpu.SemaphoreType.DMA,
    ],
    compiler_params=pltpu.CompilerParams(needs_layout_passes=False),
)
def _sc_scatter(e4_hbm, g_hbm, out_hbm, src_v, dst_v, rb0, rb1, zb,
                acc_sh, sem0, sem1):
    c = lax.axis_index("c")
    s = lax.axis_index("s")
    w = c * NS + s
    zeros16 = jnp.zeros((16,), jnp.float32)
    for r in range(ZROWS):
        for q in range(D // 16):
            zb[r, pl.ds(q * 16, 16)] = zeros16

    row0 = s * (NP // NS)

    def zero_body(t, _):
        pltpu.sync_copy(zb, acc_sh.at[pl.ds(row0 + t * ZROWS, ZROWS), :])
        return ()

    lax.fori_loop(0, (NP // NS) // ZROWS, zero_body, ())

    plsc.subcore_barrier()

    for grp in range(GROUPS):
        g_base = pl.multiple_of(grp * GROW, 8)
        pltpu.sync_copy(e4_hbm.at[0, w, pl.ds(g_base, GROW)], src_v)
        pltpu.sync_copy(e4_hbm.at[1, w, pl.ds(g_base, GROW)], dst_v)

        pltpu.async_copy(g_hbm.at[src_v.at[0]], rb0, sem0)
        pltpu.async_copy(g_hbm.at[src_v.at[1]], rb1, sem1)

        def chunk_body(jj, _):
            j0 = jj * 2
            pltpu.make_async_copy(g_hbm.at[src_v.at[j0]], rb0, sem0).wait()

            @pl.when(j0 + 2 < GROW)
            def _():
                pltpu.async_copy(g_hbm.at[src_v.at[j0 + 2]], rb0, sem0)

            pltpu.sync_copy(rb0, acc_sh.at[dst_v.at[j0]], add=True)

            pltpu.make_async_copy(g_hbm.at[src_v.at[j0 + 1]], rb1, sem1).wait()

            @pl.when(j0 + 3 < GROW)
            def _():
                pltpu.async_copy(g_hbm.at[src_v.at[j0 + 3]], rb1, sem1)

            pltpu.sync_copy(rb1, acc_sh.at[dst_v.at[j0 + 1]], add=True)
            return ()

        lax.fori_loop(0, GROW // 2, chunk_body, ())

    plsc.subcore_barrier()

    pltpu.sync_copy(acc_sh.at[pl.ds(row0, NP // NS), :],
                    out_hbm.at[c, pl.ds(row0, NP // NS), :])


def _tc1a_body(x_ref, emb_a_ref, emb_c_ref, w1_ref, hw_ref):
    xa = x_ref[:, 0:1]
    xc = x_ref[:, 1:2] + 120
    iota = lax.broadcasted_iota(jnp.int32, (1, 128), 1)
    oh = ((xa == iota) | (xc == iota)).astype(jnp.float32)
    table = jnp.concatenate(
        [emb_a_ref[...], emb_c_ref[...], jnp.zeros((4, D), jnp.float32)], axis=0)
    tw = jnp.dot(table, w1_ref[...], preferred_element_type=jnp.float32)
    hw_ref[...] = jnp.dot(oh, tw, preferred_element_type=jnp.float32)


def _tc_stage1a(x, emb_atom, emb_chiral, W1):
    return pl.pallas_call(
        _tc1a_body,
        grid=(GRID,),
        in_specs=[
            pl.BlockSpec((BLK, 2), lambda i: (i, 0)),
            pl.BlockSpec((120, D), lambda i: (0, 0)),
            pl.BlockSpec((4, D), lambda i: (0, 0)),
            pl.BlockSpec((D, D), lambda i: (0, 0)),
        ],
        out_specs=pl.BlockSpec((BLK, D), lambda i: (i, 0)),
        out_shape=jax.ShapeDtypeStruct((NP, D), jnp.float32),
    )(x, emb_atom, emb_chiral, W1)


def _tc1b_body(dp_ref, hw_ref, g1_ref, dinv_ref):
    ones = jnp.ones((NW, 1), jnp.float32)
    deg = lax.dot_general(dp_ref[...], ones, (((0,), (0,)), ((), ())),
                          preferred_element_type=jnp.float32) + 1.0
    dinv = lax.rsqrt(deg)
    dinv_ref[...] = dinv
    g1_ref[...] = dinv * hw_ref[...]


def _tc_stage1b(deg_parts, hw1):
    return pl.pallas_call(
        _tc1b_body,
        grid=(GRID,),
        in_specs=[
            pl.BlockSpec((NW, BLK), lambda i: (0, i)),
            pl.BlockSpec((BLK, D), lambda i: (i, 0)),
        ],
        out_specs=[
            pl.BlockSpec((BLK, D), lambda i: (i, 0)),
            pl.BlockSpec((BLK, 1), lambda i: (i, 0)),
        ],
        out_shape=[
            jax.ShapeDtypeStruct((NP, D), jnp.float32),
            jax.ShapeDtypeStruct((NP, 1), jnp.float32),
        ],
    )(deg_parts, hw1)


def _tc2_body(a0_ref, a1_ref, g1_ref, dinv_ref, b1_ref, w2_ref, g2_ref):
    dinv = dinv_ref[...]
    pre = dinv * (a0_ref[...] + a1_ref[...] + g1_ref[...]) + b1_ref[...]
    out1 = jnp.where(pre >= 0, pre, NEG_SLOPE * pre)
    g2_ref[...] = dinv * jnp.dot(out1, w2_ref[...],
                                 preferred_element_type=jnp.float32)


def _tc_stage2(a0, a1, g1, dinv, b1_row, W2):
    return pl.pallas_call(
        _tc2_body,
        grid=(GRID,),
        in_specs=[
            pl.BlockSpec((BLK, D), lambda i: (i, 0)),
            pl.BlockSpec((BLK, D), lambda i: (i, 0)),
            pl.BlockSpec((BLK, D), lambda i: (i, 0)),
            pl.BlockSpec((BLK, 1), lambda i: (i, 0)),
            pl.BlockSpec((1, D), lambda i: (0, 0)),
            pl.BlockSpec((D, D), lambda i: (0, 0)),
        ],
        out_specs=pl.BlockSpec((BLK, D), lambda i: (i, 0)),
        out_shape=jax.ShapeDtypeStruct((NP, D), jnp.float32),
    )(a0, a1, g1, dinv, b1_row, W2)


def _tc3_body(a0_ref, a1_ref, g2_ref, dinv_ref, b2_ref, out_ref):
    i = pl.program_id(0)
    pre = dinv_ref[...] * (a0_ref[...] + a1_ref[...] + g2_ref[...]) + b2_ref[...]
    h2 = jnp.where(pre >= 0, pre, NEG_SLOPE * pre)
    row = i * BLK + lax.broadcasted_iota(jnp.int32, (BLK, 1), 0)
    h2 = jnp.where(row < N, h2, 0.0)
    part = jnp.sum(h2, axis=0, keepdims=True) * (1.0 / N)

    @pl.when(i == 0)
    def _():
        out_ref[...] = part

    @pl.when(i != 0)
    def _():
        out_ref[...] += part


def _tc_stage3(a0, a1, g2, dinv, b2_row):
    return pl.pallas_call(
        _tc3_body,
        grid=(GRID,),
        in_specs=[
            pl.BlockSpec((BLK, D), lambda i: (i, 0)),
            pl.BlockSpec((BLK, D), lambda i: (i, 0)),
            pl.BlockSpec((BLK, D), lambda i: (i, 0)),
            pl.BlockSpec((BLK, 1), lambda i: (i, 0)),
            pl.BlockSpec((1, D), lambda i: (0, 0)),
        ],
        out_specs=pl.BlockSpec((1, D), lambda i: (0, 0)),
        out_shape=jax.ShapeDtypeStruct((1, D), jnp.float32),
    )(a0, a1, g2, dinv, b2_row)


def kernel(x, edge_index, emb_atom, emb_chiral, W1, b1, W2, b2):
    e4 = edge_index.reshape(2, NW, RPW, K)
    e3 = edge_index.reshape(2, NW, EPW // 16, 16)

    deg_parts = _sc_degree(e3)
    hw1 = _tc_stage1a(x, emb_atom, emb_chiral, W1)
    g1, dinv = _tc_stage1b(deg_parts, hw1)
    acc1 = _sc_scatter(e4, g1)
    g2 = _tc_stage2(acc1[0], acc1[1], g1, dinv, b1[None, :], W2)
    acc2 = _sc_scatter(e4, g2)
    return _tc_stage3(acc2[0], acc2[1], g2, dinv, b2[None, :])


# speedup vs baseline: 1.1058x; 1.1058x over previous
import functools

import jax
import jax.numpy as jnp
from jax import lax
from jax.experimental import pallas as pl
from jax.experimental.pallas import tpu as pltpu
from jax.experimental.pallas import tpu_sc as plsc

N = 10000
E = 320000
D = 128
NP = 10240
NC = 2
NS = 16
NW = NC * NS
K = 128
PAIRS = 13
STAGES = 3
SEDGES = PAIRS * 2 * K
WEDGES = STAGES * SEDGES
TAIL = E - NW * WEDGES
BLK = 1024
GRID = NP // BLK
ZROWS = 16
NEG_SLOPE = 0.2


def _sc_mesh():
    return plsc.VectorSubcoreMesh(core_axis_name="c", subcore_axis_name="s")


@functools.partial(
    pl.kernel,
    out_type=jax.ShapeDtypeStruct((NW, NP), jnp.float32),
    mesh=_sc_mesh(),
    scratch_types=[
        pltpu.VMEM((NP,), jnp.float32),
        pltpu.VMEM((WEDGES,), jnp.int32),
    ],
    compiler_params=pltpu.CompilerParams(needs_layout_passes=False),
)
def _sc_degree(e_hbm, out_hbm, deg_v, idx_v):
    c = lax.axis_index("c")
    s = lax.axis_index("s")
    w = c * NS + s
    zeros16 = jnp.zeros((16,), jnp.float32)
    ones16 = jnp.ones((16,), jnp.float32)

    def zero_body(i, _):
        deg_v[pl.ds(i * 16, 16)] = zeros16
        return ()

    lax.fori_loop(0, NP // 16, zero_body, (), unroll=4)

    ebase = pl.multiple_of(w * WEDGES, 128)
    pltpu.sync_copy(e_hbm.at[1, pl.ds(ebase, WEDGES)], idx_v)

    def hist_body(i, _):
        idx = idx_v[pl.ds(i * 16, 16)]
        plsc.addupdate_scatter(deg_v, [idx], ones16)
        return ()

    lax.fori_loop(0, WEDGES // 16, hist_body, (), unroll=4)

    @pl.when(w == 0)
    def _():
        pltpu.sync_copy(e_hbm.at[1, pl.ds(pl.multiple_of(NW * WEDGES, 128), TAIL)],
                        idx_v.at[pl.ds(0, TAIL)])

        def tail_body(i, _):
            idx = idx_v[pl.ds(i * 16, 16)]
            plsc.addupdate_scatter(deg_v, [idx], ones16)
            return ()

        lax.fori_loop(0, TAIL // 16, tail_body, (), unroll=4)

    pltpu.sync_copy(deg_v, out_hbm.at[w])


@functools.partial(
    pl.kernel,
    out_type=[
        jax.ShapeDtypeStruct((NP, D), jnp.float32),
        jax.ShapeDtypeStruct((NP, D), jnp.float32),
    ],
    mesh=_sc_mesh(),
    scratch_types=[
        pltpu.VMEM((SEDGES,), jnp.int32),
        pltpu.VMEM((SEDGES,), jnp.int32),
        pltpu.VMEM((K, D), jnp.float32),
        pltpu.VMEM((K, D), jnp.float32),
        pltpu.VMEM((ZROWS, D), jnp.float32),
        pltpu.VMEM_SHARED((NP, D), jnp.float32),
        pltpu.SemaphoreType.DMA,
        pltpu.SemaphoreType.DMA,
    ],
    compiler_params=pltpu.CompilerParams(needs_layout_passes=False),
)
def _sc_scatter(e_hbm, g_hbm, out0_hbm, out1_hbm, src_v, dst_v, rb0, rb1, zb,
                acc_sh, sem0, sem1):
    c = lax.axis_index("c")
    s = lax.axis_index("s")
    w = c * NS + s
    zeros16 = jnp.zeros((16,), jnp.float32)
    for r in range(ZROWS):
        for q in range(D // 16):
            zb[r, pl.ds(q * 16, 16)] = zeros16

    row0 = s * (NP // NS)

    def zero_body(t, _):
        pltpu.sync_copy(zb, acc_sh.at[pl.ds(row0 + t * ZROWS, ZROWS), :])
        return ()

    lax.fori_loop(0, (NP // NS) // ZROWS, zero_body, ())

    plsc.subcore_barrier()

    for t in range(STAGES):
        ebase = pl.multiple_of(w * WEDGES + t * SEDGES, 128)
        pltpu.sync_copy(e_hbm.at[0, pl.ds(ebase, SEDGES)], src_v)
        pltpu.sync_copy(e_hbm.at[1, pl.ds(ebase, SEDGES)], dst_v)

        pltpu.async_copy(g_hbm.at[src_v.at[pl.ds(0, K)]], rb0, sem0)
        pltpu.async_copy(g_hbm.at[src_v.at[pl.ds(K, K)]], rb1, sem1)

        def pair_body(j, _):
            o0 = pl.multiple_of(j * 2 * K, 128)
            o1 = pl.multiple_of(j * 2 * K + K, 128)
            o2 = pl.multiple_of(j * 2 * K + 2 * K, 128)
            o3 = pl.multiple_of(j * 2 * K + 3 * K, 128)
            pltpu.make_async_copy(g_hbm.at[src_v.at[pl.ds(o0, K)]], rb0,
                                  sem0).wait()

            @pl.when(j + 1 < PAIRS)
            def _():
                pltpu.async_copy(g_hbm.at[src_v.at[pl.ds(o2, K)]], rb0, sem0)

            pltpu.sync_copy(rb0, acc_sh.at[dst_v.at[pl.ds(o0, K)]], add=True)

            pltpu.make_async_copy(g_hbm.at[src_v.at[pl.ds(o1, K)]], rb1,
                                  sem1).wait()

            @pl.when(j + 1 < PAIRS)
            def _():
                pltpu.async_copy(g_hbm.at[src_v.at[pl.ds(o3, K)]], rb1, sem1)

            pltpu.sync_copy(rb1, acc_sh.at[dst_v.at[pl.ds(o1, K)]], add=True)
            return ()

        lax.fori_loop(0, PAIRS, pair_body, ())

    @pl.when(w < TAIL // K)
    def _():
        toff = pl.multiple_of(NW * WEDGES + w * K, 128)
        pltpu.sync_copy(e_hbm.at[0, pl.ds(toff, K)], src_v.at[pl.ds(0, K)])
        pltpu.sync_copy(e_hbm.at[1, pl.ds(toff, K)], dst_v.at[pl.ds(0, K)])
        pltpu.sync_copy(g_hbm.at[src_v.at[pl.ds(0, K)]], rb0)
        pltpu.sync_copy(rb0, acc_sh.at[dst_v.at[pl.ds(0, K)]], add=True)

    plsc.subcore_barrier()

    rows = acc_sh.at[pl.ds(row0, NP // NS), :]

    @pl.when(c == 0)
    def _():
        pltpu.sync_copy(rows, out0_hbm.at[pl.ds(row0, NP // NS), :])

    @pl.when(c == 1)
    def _():
        pltpu.sync_copy(rows, out1_hbm.at[pl.ds(row0, NP // NS), :])


def _tc1a_body(x_ref, emb_a_ref, emb_c_ref, w1_ref, hw_ref):
    xa = x_ref[:, 0:1]
    xc = x_ref[:, 1:2] + 120
    iota = lax.broadcasted_iota(jnp.int32, (1, 128), 1)
    oh = ((xa == iota) | (xc == iota)).astype(jnp.float32)
    table = jnp.concatenate(
        [emb_a_ref[...], emb_c_ref[...], jnp.zeros((4, D), jnp.float32)], axis=0)
    tw = jnp.dot(table, w1_ref[...], preferred_element_type=jnp.float32)
    hw_ref[...] = jnp.dot(oh, tw, preferred_element_type=jnp.float32)


def _tc_stage1a(x, emb_atom, emb_chiral, W1):
    return pl.pallas_call(
        _tc1a_body,
        grid=(GRID,),
        in_specs=[
            pl.BlockSpec((BLK, 2), lambda i: (i, 0)),
            pl.BlockSpec((120, D), lambda i: (0, 0)),
            pl.BlockSpec((4, D), lambda i: (0, 0)),
            pl.BlockSpec((D, D), lambda i: (0, 0)),
        ],
        out_specs=pl.BlockSpec((BLK, D), lambda i: (i, 0)),
        out_shape=jax.ShapeDtypeStruct((NP, D), jnp.float32),
    )(x, emb_atom, emb_chiral, W1)


def _tc1b_body(dp_ref, hw_ref, g1_ref, dinv_ref):
    ones = jnp.ones((NW, 1), jnp.float32)
    deg = lax.dot_general(dp_ref[...], ones, (((0,), (0,)), ((), ())),
                          preferred_element_type=jnp.float32) + 1.0
    dinv = lax.rsqrt(deg)
    dinv_ref[...] = dinv
    g1_ref[...] = dinv * hw_ref[...]


def _tc_stage1b(deg_parts, hw1):
    return pl.pallas_call(
        _tc1b_body,
        grid=(GRID,),
        in_specs=[
            pl.BlockSpec((NW, BLK), lambda i: (0, i)),
            pl.BlockSpec((BLK, D), lambda i: (i, 0)),
        ],
        out_specs=[
            pl.BlockSpec((BLK, D), lambda i: (i, 0)),
            pl.BlockSpec((BLK, 1), lambda i: (i, 0)),
        ],
        out_shape=[
            jax.ShapeDtypeStruct((NP, D), jnp.float32),
            jax.ShapeDtypeStruct((NP, 1), jnp.float32),
        ],
    )(deg_parts, hw1)


def _tc2_body(a0_ref, a1_ref, g1_ref, dinv_ref, b1_ref, w2_ref, g2_ref):
    dinv = dinv_ref[...]
    pre = dinv * (a0_ref[...] + a1_ref[...] + g1_ref[...]) + b1_ref[...]
    out1 = jnp.where(pre >= 0, pre, NEG_SLOPE * pre)
    g2_ref[...] = dinv * jnp.dot(out1, w2_ref[...],
                                 preferred_element_type=jnp.float32)


def _tc_stage2(a0, a1, g1, dinv, b1_row, W2):
    return pl.pallas_call(
        _tc2_body,
        grid=(GRID,),
        in_specs=[
            pl.BlockSpec((BLK, D), lambda i: (i, 0)),
            pl.BlockSpec((BLK, D), lambda i: (i, 0)),
            pl.BlockSpec((BLK, D), lambda i: (i, 0)),
            pl.BlockSpec((BLK, 1), lambda i: (i, 0)),
            pl.BlockSpec((1, D), lambda i: (0, 0)),
            pl.BlockSpec((D, D), lambda i: (0, 0)),
        ],
        out_specs=pl.BlockSpec((BLK, D), lambda i: (i, 0)),
        out_shape=jax.ShapeDtypeStruct((NP, D), jnp.float32),
    )(a0, a1, g1, dinv, b1_row, W2)


def _tc3_body(a0_ref, a1_ref, g2_ref, dinv_ref, b2_ref, out_ref):
    i = pl.program_id(0)
    pre = dinv_ref[...] * (a0_ref[...] + a1_ref[...] + g2_ref[...]) + b2_ref[...]
    h2 = jnp.where(pre >= 0, pre, NEG_SLOPE * pre)
    row = i * BLK + lax.broadcasted_iota(jnp.int32, (BLK, 1), 0)
    h2 = jnp.where(row < N, h2, 0.0)
    part = jnp.sum(h2, axis=0, keepdims=True) * (1.0 / N)

    @pl.when(i == 0)
    def _():
        out_ref[...] = part

    @pl.when(i != 0)
    def _():
        out_ref[...] += part


def _tc_stage3(a0, a1, g2, dinv, b2_row):
    return pl.pallas_call(
        _tc3_body,
        grid=(GRID,),
        in_specs=[
            pl.BlockSpec((BLK, D), lambda i: (i, 0)),
            pl.BlockSpec((BLK, D), lambda i: (i, 0)),
            pl.BlockSpec((BLK, D), lambda i: (i, 0)),
            pl.BlockSpec((BLK, 1), lambda i: (i, 0)),
            pl.BlockSpec((1, D), lambda i: (0, 0)),
        ],
        out_specs=pl.BlockSpec((1, D), lambda i: (0, 0)),
        out_shape=jax.ShapeDtypeStruct((1, D), jnp.float32),
    )(a0, a1, g2, dinv, b2_row)


def kernel(x, edge_index, emb_atom, emb_chiral, W1, b1, W2, b2):
    deg_parts = _sc_degree(edge_index)
    hw1 = _tc_stage1a(x, emb_atom, emb_chiral, W1)
    g1, dinv = _tc_stage1b(deg_parts, hw1)
    a1, a1b = _sc_scatter(edge_index, g1)
    g2 = _tc_stage2(a1, a1b, g1, dinv, b1[None, :], W2)
    a2, a2b = _sc_scatter(edge_index, g2)
    return _tc_stage3(a2, a2b, g2, dinv, b2[None, :])


# speedup vs baseline: 1.1750x; 1.0626x over previous
import functools

import jax
import jax.numpy as jnp
from jax import lax
from jax.experimental import pallas as pl
from jax.experimental.pallas import tpu as pltpu
from jax.experimental.pallas import tpu_sc as plsc

N = 10000
E = 320000
D = 128
NP = 10240
NC = 2
NS = 16
NW = NC * NS
K = 128
PAIRS = 13
STAGES = 3
SEDGES = PAIRS * 2 * K
WEDGES = STAGES * SEDGES
TAIL = E - NW * WEDGES
BLK = 2048
GRID = NP // BLK
ZROWS = 16
NEG_SLOPE = 0.2


def _sc_mesh():
    return plsc.VectorSubcoreMesh(core_axis_name="c", subcore_axis_name="s")


@functools.partial(
    pl.kernel,
    out_type=jax.ShapeDtypeStruct((NW, NP), jnp.float32),
    mesh=_sc_mesh(),
    scratch_types=[
        pltpu.VMEM((NP,), jnp.float32),
        pltpu.VMEM((WEDGES,), jnp.int32),
    ],
    compiler_params=pltpu.CompilerParams(needs_layout_passes=False),
)
def _sc_degree(e_hbm, out_hbm, deg_v, idx_v):
    c = lax.axis_index("c")
    s = lax.axis_index("s")
    w = c * NS + s
    zeros16 = jnp.zeros((16,), jnp.float32)
    ones16 = jnp.ones((16,), jnp.float32)

    def zero_body(i, _):
        deg_v[pl.ds(i * 16, 16)] = zeros16
        return ()

    lax.fori_loop(0, NP // 16, zero_body, (), unroll=4)

    ebase = pl.multiple_of(w * WEDGES, 128)
    pltpu.sync_copy(e_hbm.at[1, pl.ds(ebase, WEDGES)], idx_v)

    def hist_body(i, _):
        idx = idx_v[pl.ds(i * 16, 16)]
        plsc.addupdate_scatter(deg_v, [idx], ones16)
        return ()

    lax.fori_loop(0, WEDGES // 16, hist_body, (), unroll=4)

    @pl.when(w == 0)
    def _():
        pltpu.sync_copy(e_hbm.at[1, pl.ds(pl.multiple_of(NW * WEDGES, 128), TAIL)],
                        idx_v.at[pl.ds(0, TAIL)])

        def tail_body(i, _):
            idx = idx_v[pl.ds(i * 16, 16)]
            plsc.addupdate_scatter(deg_v, [idx], ones16)
            return ()

        lax.fori_loop(0, TAIL // 16, tail_body, (), unroll=4)

    pltpu.sync_copy(deg_v, out_hbm.at[w])


@functools.partial(
    pl.kernel,
    out_type=[
        jax.ShapeDtypeStruct((NP, D), jnp.float32),
        jax.ShapeDtypeStruct((NP, D), jnp.float32),
    ],
    mesh=_sc_mesh(),
    scratch_types=[
        pltpu.VMEM((2, SEDGES), jnp.int32),
        pltpu.VMEM((2, SEDGES), jnp.int32),
        pltpu.VMEM((K, D), jnp.float32),
        pltpu.VMEM((K, D), jnp.float32),
        pltpu.VMEM((ZROWS, D), jnp.float32),
        pltpu.VMEM_SHARED((NP, D), jnp.float32),
        pltpu.SemaphoreType.DMA,
        pltpu.SemaphoreType.DMA,
        pltpu.SemaphoreType.DMA,
        pltpu.SemaphoreType.DMA,
    ],
    compiler_params=pltpu.CompilerParams(needs_layout_passes=False),
)
def _sc_scatter(e_hbm, g_hbm, out0_hbm, out1_hbm, src_v, dst_v, rb0, rb1, zb,
                acc_sh, sem0, sem1, isem0, isem1):
    c = lax.axis_index("c")
    s = lax.axis_index("s")
    w = c * NS + s
    isems = (isem0, isem1)

    def stage_descs(t):
        p = t % 2
        ebase = pl.multiple_of(w * WEDGES + t * SEDGES, 128)
        return (pltpu.make_async_copy(e_hbm.at[0, pl.ds(ebase, SEDGES)],
                                      src_v.at[p], isems[p]),
                pltpu.make_async_copy(e_hbm.at[1, pl.ds(ebase, SEDGES)],
                                      dst_v.at[p], isems[p]))

    for d in stage_descs(0):
        d.start()

    zeros16 = jnp.zeros((16,), jnp.float32)
    for r in range(ZROWS):
        for q in range(D // 16):
            zb[r, pl.ds(q * 16, 16)] = zeros16

    row0 = s * (NP // NS)

    def zero_body(t, _):
        pltpu.sync_copy(zb, acc_sh.at[pl.ds(row0 + t * ZROWS, ZROWS), :])
        return ()

    lax.fori_loop(0, (NP // NS) // ZROWS, zero_body, ())

    plsc.subcore_barrier()

    for t in range(STAGES):
        p = t % 2
        for d in stage_descs(t):
            d.wait()
        if t + 1 < STAGES:
            for d in stage_descs(t + 1):
                d.start()

        sv = src_v.at[p]
        dv = dst_v.at[p]
        pltpu.async_copy(g_hbm.at[sv.at[pl.ds(0, K)]], rb0, sem0)
        pltpu.async_copy(g_hbm.at[sv.at[pl.ds(K, K)]], rb1, sem1)

        def pair_body(j, _):
            o0 = pl.multiple_of(j * 2 * K, 128)
            o1 = pl.multiple_of(j * 2 * K + K, 128)
            o2 = pl.multiple_of(j * 2 * K + 2 * K, 128)
            o3 = pl.multiple_of(j * 2 * K + 3 * K, 128)
            pltpu.make_async_copy(g_hbm.at[sv.at[pl.ds(o0, K)]], rb0,
                                  sem0).wait()

            @pl.when(j + 1 < PAIRS)
            def _():
                pltpu.async_copy(g_hbm.at[sv.at[pl.ds(o2, K)]], rb0, sem0)

            pltpu.sync_copy(rb0, acc_sh.at[dv.at[pl.ds(o0, K)]], add=True)

            pltpu.make_async_copy(g_hbm.at[sv.at[pl.ds(o1, K)]], rb1,
                                  sem1).wait()

            @pl.when(j + 1 < PAIRS)
            def _():
                pltpu.async_copy(g_hbm.at[sv.at[pl.ds(o3, K)]], rb1, sem1)

            pltpu.sync_copy(rb1, acc_sh.at[dv.at[pl.ds(o1, K)]], add=True)
            return ()

        lax.fori_loop(0, PAIRS, pair_body, ())

    @pl.when(w < TAIL // K)
    def _():
        toff = pl.multiple_of(NW * WEDGES + w * K, 128)
        pltpu.sync_copy(e_hbm.at[0, pl.ds(toff, K)], src_v.at[0, pl.ds(0, K)])
        pltpu.sync_copy(e_hbm.at[1, pl.ds(toff, K)], dst_v.at[0, pl.ds(0, K)])
        pltpu.sync_copy(g_hbm.at[src_v.at[0, pl.ds(0, K)]], rb0)
        pltpu.sync_copy(rb0, acc_sh.at[dst_v.at[0, pl.ds(0, K)]], add=True)

    plsc.subcore_barrier()

    rows = acc_sh.at[pl.ds(row0, NP // NS), :]

    @pl.when(c == 0)
    def _():
        pltpu.sync_copy(rows, out0_hbm.at[pl.ds(row0, NP // NS), :])

    @pl.when(c == 1)
    def _():
        pltpu.sync_copy(rows, out1_hbm.at[pl.ds(row0, NP // NS), :])


def _tc1a_body(x_ref, emb_a_ref, emb_c_ref, w1_ref, hw_ref):
    xa = x_ref[:, 0:1]
    xc = x_ref[:, 1:2] + 120
    iota = lax.broadcasted_iota(jnp.int32, (1, 128), 1)
    oh = ((xa == iota) | (xc == iota)).astype(jnp.float32)
    table = jnp.concatenate(
        [emb_a_ref[...], emb_c_ref[...], jnp.zeros((4, D), jnp.float32)], axis=0)
    tw = jnp.dot(table, w1_ref[...], preferred_element_type=jnp.float32)
    hw_ref[...] = jnp.dot(oh, tw, preferred_element_type=jnp.float32)


def _tc_stage1a(x, emb_atom, emb_chiral, W1):
    return pl.pallas_call(
        _tc1a_body,
        grid=(GRID,),
        in_specs=[
            pl.BlockSpec((BLK, 2), lambda i: (i, 0)),
            pl.BlockSpec((120, D), lambda i: (0, 0)),
            pl.BlockSpec((4, D), lambda i: (0, 0)),
            pl.BlockSpec((D, D), lambda i: (0, 0)),
        ],
        out_specs=pl.BlockSpec((BLK, D), lambda i: (i, 0)),
        out_shape=jax.ShapeDtypeStruct((NP, D), jnp.float32),
    )(x, emb_atom, emb_chiral, W1)


def _tc1b_body(dp_ref, hw_ref, g1_ref, dinv_ref):
    ones = jnp.ones((NW, 1), jnp.float32)
    deg = lax.dot_general(dp_ref[...], ones, (((0,), (0,)), ((), ())),
                          preferred_element_type=jnp.float32) + 1.0
    dinv = lax.rsqrt(deg)
    dinv_ref[...] = dinv
    g1_ref[...] = dinv * hw_ref[...]


def _tc_stage1b(deg_parts, hw1):
    return pl.pallas_call(
        _tc1b_body,
        grid=(GRID,),
        in_specs=[
            pl.BlockSpec((NW, BLK), lambda i: (0, i)),
            pl.BlockSpec((BLK, D), lambda i: (i, 0)),
        ],
        out_specs=[
            pl.BlockSpec((BLK, D), lambda i: (i, 0)),
            pl.BlockSpec((BLK, 1), lambda i: (i, 0)),
        ],
        out_shape=[
            jax.ShapeDtypeStruct((NP, D), jnp.float32),
            jax.ShapeDtypeStruct((NP, 1), jnp.float32),
        ],
    )(deg_parts, hw1)


def _tc2_body(a0_ref, a1_ref, g1_ref, dinv_ref, b1_ref, w2_ref, g2_ref):
    dinv = dinv_ref[...]
    pre = dinv * (a0_ref[...] + a1_ref[...] + g1_ref[...]) + b1_ref[...]
    out1 = jnp.where(pre >= 0, pre, NEG_SLOPE * pre)
    g2_ref[...] = dinv * jnp.dot(out1, w2_ref[...],
                                 preferred_element_type=jnp.float32)


def _tc_stage2(a0, a1, g1, dinv, b1_row, W2):
    return pl.pallas_call(
        _tc2_body,
        grid=(GRID,),
        in_specs=[
            pl.BlockSpec((BLK, D), lambda i: (i, 0)),
            pl.BlockSpec((BLK, D), lambda i: (i, 0)),
            pl.BlockSpec((BLK, D), lambda i: (i, 0)),
            pl.BlockSpec((BLK, 1), lambda i: (i, 0)),
            pl.BlockSpec((1, D), lambda i: (0, 0)),
            pl.BlockSpec((D, D), lambda i: (0, 0)),
        ],
        out_specs=pl.BlockSpec((BLK, D), lambda i: (i, 0)),
        out_shape=jax.ShapeDtypeStruct((NP, D), jnp.float32),
    )(a0, a1, g1, dinv, b1_row, W2)


def _tc3_body(a0_ref, a1_ref, g2_ref, dinv_ref, b2_ref, out_ref):
    i = pl.program_id(0)
    pre = dinv_ref[...] * (a0_ref[...] + a1_ref[...] + g2_ref[...]) + b2_ref[...]
    h2 = jnp.where(pre >= 0, pre, NEG_SLOPE * pre)
    row = i * BLK + lax.broadcasted_iota(jnp.int32, (BLK, 1), 0)
    h2 = jnp.where(row < N, h2, 0.0)
    part = jnp.sum(h2, axis=0, keepdims=True) * (1.0 / N)

    @pl.when(i == 0)
    def _():
        out_ref[...] = part

    @pl.when(i != 0)
    def _():
        out_ref[...] += part


def _tc_stage3(a0, a1, g2, dinv, b2_row):
    return pl.pallas_call(
        _tc3_body,
        grid=(GRID,),
        in_specs=[
            pl.BlockSpec((BLK, D), lambda i: (i, 0)),
            pl.BlockSpec((BLK, D), lambda i: (i, 0)),
            pl.BlockSpec((BLK, D), lambda i: (i, 0)),
            pl.BlockSpec((BLK, 1), lambda i: (i, 0)),
            pl.BlockSpec((1, D), lambda i: (0, 0)),
        ],
        out_specs=pl.BlockSpec((1, D), lambda i: (0, 0)),
        out_shape=jax.ShapeDtypeStruct((1, D), jnp.float32),
    )(a0, a1, g2, dinv, b2_row)


def kernel(x, edge_index, emb_atom, emb_chiral, W1, b1, W2, b2):
    deg_parts = _sc_degree(edge_index)
    hw1 = _tc_stage1a(x, emb_atom, emb_chiral, W1)
    g1, dinv = _tc_stage1b(deg_parts, hw1)
    a1, a1b = _sc_scatter(edge_index, g1)
    g2 = _tc_stage2(a1, a1b, g1, dinv, b1[None, :], W2)
    a2, a2b = _sc_scatter(edge_index, g2)
    return _tc_stage3(a2, a2b, g2, dinv, b2[None, :])
